# baseline (speedup 1.0000x reference)
import functools

import jax
import jax.numpy as jnp
from jax.experimental import pallas as pl

F32 = jnp.float32
BF16 = jnp.bfloat16
HI = jax.lax.Precision.HIGHEST
DOWN_T, STRIDE_T, DEPTH, DGR = 3, 2, 3, 3
N, T0, CIN, W, CODE, NB_CODE = 64, 64, 263, 512, 512, 1024

ENC_C3_BF = (False, False, True)
ENC_C1_BF = {0: (True, True, False), 1: (True, True, False),
             2: (True, True, True)}
DEC_C3_BF = (False, False, False)
DEC_C1_BF = (True, True, True)


def _mm3(a, b):
    return jax.lax.dot_general(a, b, (((2,), (0,)), ((), ())),
                               preferred_element_type=F32)


def _mm_hi(a, b):
    return jax.lax.dot_general(a, b, (((1,), (0,)), ((), ())),
                               precision=HI, preferred_element_type=F32)


def _relu(x):
    return jnp.maximum(x, 0.0)


def _csame(h, w_ref, b_ref, dil, widx, bidx, lhs_bf):
    n, t, c = h.shape
    z = jnp.zeros((n, dil, c), F32)
    hp = jnp.concatenate([z, h, z], axis=1)
    if lhs_bf:
        hp = hp.astype(BF16)
    out = b_ref[pl.ds(bidx, 1), :][None]
    for j in range(3):
        out = out + _mm3(hp[:, j * dil:j * dil + t, :], w_ref[widx + j])
    return out


def _res_chain(h, w3_ref, b3_ref, w1_ref, b1_ref, dils, c3_bf, c1_bf):
    for u, dil in enumerate(dils):
        r = _relu(h)
        r = _csame(r, w3_ref, b3_ref, dil, 3 * u, u, c3_bf[u])
        r = _relu(r)
        if c1_bf[u]:
            r = r.astype(BF16)
        r = b1_ref[pl.ds(u, 1), :][None] + _mm3(r, w1_ref[u])
        h = h + r
    return h


def _conv0_body(x_ref, w_ref, b_ref, o_ref, *, relu_out, lhs_bf):
    out = _csame(x_ref[...], w_ref, b_ref, 1, 0, 0, lhs_bf)
    o_ref[...] = _relu(out) if relu_out else out


def _enc_block_body(xp_ref, wd_ref, bd_ref, w3_ref, b3_ref, w1_ref, b1_ref,
                    o_ref, *, tout):
    xp = xp_ref[...].astype(BF16)
    ev = xp[:, :, :W]
    od = xp[:, :, W:]
    h = (bd_ref[pl.ds(0, 1), :][None]
         + _mm3(ev[:, :tout, :], wd_ref[0])
         + _mm3(od[:, :tout, :], wd_ref[1])
         + _mm3(ev[:, 1:tout + 1, :], wd_ref[2])
         + _mm3(od[:, 1:tout + 1, :], wd_ref[3]))
    o_ref[...] = _res_chain(h, w3_ref, b3_ref, w1_ref, b1_ref, (1, 3),
                            ENC_C3_BF, (True, True))


def _u3fin_body(h_ref, r9_ref, w1_ref, b1_ref, o_ref, *, c1bf):
    r = _relu(r9_ref[...])
    if c1bf:
        r = r.astype(BF16)
    r = b1_ref[pl.ds(0, 1), :][None] + _mm3(r, w1_ref[0])
    o_ref[...] = h_ref[...] + r


def _dec_block_body(x_ref, w3_ref, b3_ref, w1_ref, b1_ref, wu_ref, bu_ref,
                    o_ref):
    h = _res_chain(x_ref[...], w3_ref, b3_ref, w1_ref, b1_ref, (9, 3, 1),
                   DEC_C3_BF, DEC_C1_BF)
    n, t, c = h.shape
    z = jnp.zeros((n, 1, c), F32)
    hp = jnp.concatenate([z, h, z], axis=1).astype(BF16)
    bu = bu_ref[pl.ds(0, 1), :][None]
    c0 = hp[:, 0:t, :]
    c1 = hp[:, 1:t + 1, :]
    c2 = hp[:, 2:t + 2, :]
    ev = bu + _mm3(c0, wu_ref[0]) + _mm3(c1, wu_ref[1]) + _mm3(c1, wu_ref[2])
    od = bu + _mm3(c1, wu_ref[0]) + _mm3(c1, wu_ref[1]) + _mm3(c2, wu_ref[2])
    o_ref[...] = jnp.concatenate([ev, od], axis=-1)


def _final_body(x_ref, wa_ref, ba_ref, wb_ref, bb_ref, o_ref):
    h = _relu(_csame(x_ref[...], wa_ref, ba_ref, 1, 0, 0, False))
    o_ref[...] = _csame(h, wb_ref, bb_ref, 1, 0, 0, True)


def _vq_body(x_ref, w_ref, b_ref, cb_ref, n2_ref, qh_ref, idx_ref, loss_ref):
    h = _csame(x_ref[...], w_ref, b_ref, 1, 0, 0, True)
    rows = N * (T0 // 8)
    flat = h.reshape(rows, CODE)
    cb = cb_ref[...]
    s = jnp.sum(flat * flat, axis=-1, keepdims=True)
    mm = jax.lax.dot_general(flat.astype(BF16), cb, (((1,), (1,)), ((), ())),
                             preferred_element_type=F32)
    d = s - 2.0 * mm + n2_ref[...]
    dmin = jnp.min(d, axis=-1, keepdims=True)
    ii = jax.lax.broadcasted_iota(jnp.int32, d.shape, 1)
    cand = jnp.where(d <= dmin, ii, NB_CODE)
    idx = jnp.min(cand, axis=-1, keepdims=True)
    oh = (ii == idx).astype(F32)
    q = _mm_hi(oh, cb)
    loss_ref[...] = jnp.mean((q - flat) ** 2).reshape(1, 1)
    qh_ref[...] = q.reshape(N, T0 // 8, CODE)
    idx_ref[...] = idx


def _full_spec(shape):
    return pl.BlockSpec(shape, lambda i: (0,) * len(shape))


def _batch_spec(shape, nb):
    blk = (nb,) + shape[1:]
    return pl.BlockSpec(blk, lambda i: (i,) + (0,) * (len(shape) - 1))


def _stage(body, x, weights, out_shapes, nb, multi_out=False, extra=()):
    grid = (x.shape[0] // nb,)
    in_specs = ([_batch_spec(x.shape, nb)]
                + [_batch_spec(e.shape, nb) for e in extra]
                + [_full_spec(w.shape) for w in weights])
    if multi_out:
        out_specs = [_full_spec(s.shape) for s in out_shapes]
        out_specs[0] = _batch_spec(out_shapes[0].shape, nb)
    else:
        out_specs = _batch_spec(out_shapes.shape, nb)
    return pl.pallas_call(
        body, grid=grid, in_specs=in_specs, out_specs=out_specs,
        out_shape=out_shapes)(x, *extra, *weights)


def _wt(w, dtype=F32):
    return jnp.transpose(w, (2, 1, 0)).astype(dtype)


def _pair(h):
    n, t, c = h.shape
    hp = jnp.pad(h, ((0, 0), (1, 1), (0, 0)))
    return hp.reshape(n, (t + 2) // 2, 2 * c)


def _res_weights(res):
    w3 = jnp.concatenate([_wt(c1[0]) for (c1, c2) in res], axis=0)
    b3 = jnp.stack([c1[1] for (c1, c2) in res], axis=0)
    w1 = jnp.stack([_wt(c2[0], BF16)[0] for (c1, c2) in res], axis=0)
    b1 = jnp.stack([c2[1] for (c1, c2) in res], axis=0)
    return w3, b3, w1, b1


def _xconv_d9(h_ntc, wb):
    w, b = wb
    h = h_ntc.transpose(0, 2, 1)
    r = _relu(h)
    out = jax.lax.conv_general_dilated(
        r, w, window_strides=(1,), padding=[(9, 9)], rhs_dilation=(9,),
        dimension_numbers=('NCH', 'OIH', 'NCH'))
    return (out + b[None, :, None]).transpose(0, 2, 1)


def kernel(x, enc_params, dec_params, codebook):
    x = x.astype(F32)

    w0, b0 = enc_params['conv0']
    h = jax.lax.conv_general_dilated(
        x.transpose(0, 2, 1), w0, window_strides=(1,), padding=[(1, 1)],
        dimension_numbers=('NCH', 'OIH', 'NCH'))
    h = _relu(h + b0[None, :, None]).transpose(0, 2, 1)
    t = T0
    for bi, blk in enumerate(enc_params['downs']):
        t //= 2
        wd = _wt(blk['down'][0], BF16)
        bd = blk['down'][1][None]
        w3, b3, w1, b1 = _res_weights(blk['res'])
        h2 = _stage(functools.partial(_enc_block_body, tout=t),
                    _pair(h), (wd, bd, w3, b3, w1, b1),
                    jax.ShapeDtypeStruct((N, t, W), F32), nb=max(8, 512 // t))
        r9 = _xconv_d9(h2, blk['res'][2][0])
        h = _stage(functools.partial(_u3fin_body, c1bf=ENC_C1_BF[bi][2]),
                   h2, (w1[2:3], b1[2:3]),
                   jax.ShapeDtypeStruct((N, t, W), F32),
                   nb=max(8, 512 // t), extra=(r9,))

    rows = N * t
    n2 = jnp.sum(codebook * codebook, axis=-1)[None, :]
    qh, idx, loss = _stage(
        _vq_body, h,
        (_wt(enc_params['conv_out'][0]), enc_params['conv_out'][1][None],
         codebook, n2),
        (jax.ShapeDtypeStruct((N, t, CODE), F32),
         jax.ShapeDtypeStruct((rows, 1), jnp.int32),
         jax.ShapeDtypeStruct((1, 1), F32)),
        nb=N, multi_out=True)

    h = _stage(functools.partial(_conv0_body, relu_out=True, lhs_bf=False),
               qh, (_wt(dec_params['conv0'][0]), dec_params['conv0'][1][None]),
               jax.ShapeDtypeStruct((N, t, W), F32), nb=N)
    for blk in dec_params['ups']:
        w3, b3, w1, b1 = _res_weights(blk['res'])
        wu = _wt(blk['up_conv'][0])
        bu = blk['up_conv'][1][None]
        hp = _stage(_dec_block_body, h, (w3, b3, w1, b1, wu, bu),
                    jax.ShapeDtypeStruct((N, t, 2 * W), F32),
                    nb=max(8, 512 // t))
        t *= 2
        h = hp.reshape(N, t, W)

    out = _stage(_final_body, h,
                 (_wt(dec_params['conv1'][0]), dec_params['conv1'][1][None],
                  _wt(dec_params['conv_final'][0]),
                  dec_params['conv_final'][1][None]),
                 jax.ShapeDtypeStruct((N, T0, CIN), F32), nb=8)

    return out, idx.reshape(N, T0 // 8), loss.reshape(()), qh


# speedup vs baseline: 1.0072x; 1.0072x over previous
import functools

import jax
import jax.numpy as jnp
from jax.experimental import pallas as pl

F32 = jnp.float32
BF16 = jnp.bfloat16
HI = jax.lax.Precision.HIGHEST
DOWN_T, STRIDE_T, DEPTH, DGR = 3, 2, 3, 3
N, T0, CIN, W, CODE, NB_CODE = 64, 64, 263, 512, 512, 1024

ENC_C3_BF = (False, False, True)
ENC_C1_BF = {0: (True, True, False), 1: (True, True, False),
             2: (True, True, True)}
DEC_C3_BF = (False, False, False)
DEC_C1_BF = (True, True, True)


def _mm3(a, b):
    return jax.lax.dot_general(a, b, (((2,), (0,)), ((), ())),
                               preferred_element_type=F32)


def _mm_hi(a, b):
    return jax.lax.dot_general(a, b, (((1,), (0,)), ((), ())),
                               precision=HI, preferred_element_type=F32)


def _relu(x):
    return jnp.maximum(x, 0.0)


def _csame(h, w_ref, b_ref, dil, widx, bidx, lhs_bf):
    n, t, c = h.shape
    z = jnp.zeros((n, dil, c), F32)
    hp = jnp.concatenate([z, h, z], axis=1)
    if lhs_bf:
        hp = hp.astype(BF16)
    out = b_ref[pl.ds(bidx, 1), :][None]
    for j in range(3):
        out = out + _mm3(hp[:, j * dil:j * dil + t, :], w_ref[widx + j])
    return out


def _res_chain(h, w3_ref, b3_ref, w1_ref, b1_ref, dils, c3_bf, c1_bf):
    for u, dil in enumerate(dils):
        r = _relu(h)
        r = _csame(r, w3_ref, b3_ref, dil, 3 * u, u, c3_bf[u])
        r = _relu(r)
        if c1_bf[u]:
            r = r.astype(BF16)
        r = b1_ref[pl.ds(u, 1), :][None] + _mm3(r, w1_ref[u])
        h = h + r
    return h


def _conv0_body(x_ref, w_ref, b_ref, o_ref, *, relu_out, lhs_bf):
    out = _csame(x_ref[...], w_ref, b_ref, 1, 0, 0, lhs_bf)
    o_ref[...] = _relu(out) if relu_out else out


def _enc_block_body(xp_ref, wd_ref, bd_ref, w3_ref, b3_ref, w1_ref, b1_ref,
                    o_ref, *, tout):
    xp = xp_ref[...].astype(BF16)
    ev = xp[:, :, :W]
    od = xp[:, :, W:]
    h = (bd_ref[pl.ds(0, 1), :][None]
         + _mm3(ev[:, :tout, :], wd_ref[0])
         + _mm3(od[:, :tout, :], wd_ref[1])
         + _mm3(ev[:, 1:tout + 1, :], wd_ref[2])
         + _mm3(od[:, 1:tout + 1, :], wd_ref[3]))
    o_ref[...] = _res_chain(h, w3_ref, b3_ref, w1_ref, b1_ref, (1, 3),
                            ENC_C3_BF, (True, True))


def _u3fin_body(h_ref, r9_ref, w1_ref, b1_ref, o_ref, *, c1bf):
    r = _relu(r9_ref[...])
    if c1bf:
        r = r.astype(BF16)
    r = b1_ref[pl.ds(0, 1), :][None] + _mm3(r, w1_ref[0])
    o_ref[...] = h_ref[...] + r


def _dec_block_body(x_ref, w3_ref, b3_ref, w1_ref, b1_ref, wu_ref, bu_ref,
                    o_ref):
    h = _res_chain(x_ref[...], w3_ref, b3_ref, w1_ref, b1_ref, (9, 3, 1),
                   DEC_C3_BF, DEC_C1_BF)
    n, t, c = h.shape
    z = jnp.zeros((n, 1, c), F32)
    hp = jnp.concatenate([z, h, z], axis=1).astype(BF16)
    bu = bu_ref[pl.ds(0, 1), :][None]
    c0 = hp[:, 0:t, :]
    c1 = hp[:, 1:t + 1, :]
    c2 = hp[:, 2:t + 2, :]
    ev = bu + _mm3(c0, wu_ref[0]) + _mm3(c1, wu_ref[1]) + _mm3(c1, wu_ref[2])
    od = bu + _mm3(c1, wu_ref[0]) + _mm3(c1, wu_ref[1]) + _mm3(c2, wu_ref[2])
    o_ref[...] = jnp.concatenate([ev, od], axis=-1)


def _final_body(x_ref, wa_ref, ba_ref, wb_ref, bb_ref, o_ref):
    h = _relu(_csame(x_ref[...], wa_ref, ba_ref, 1, 0, 0, False))
    o_ref[...] = _csame(h, wb_ref, bb_ref, 1, 0, 0, True)


def _vq_body(x_ref, r9_ref, w1_ref, b1_ref, w_ref, b_ref, cb_ref, n2_ref,
             wd0_ref, bd0_ref, qh_ref, idx_ref, loss_ref, hd_ref):
    r = _relu(r9_ref[...]).astype(BF16)
    r = b1_ref[pl.ds(0, 1), :][None] + _mm3(r, w1_ref[0])
    h3 = x_ref[...] + r
    h = _csame(h3, w_ref, b_ref, 1, 0, 0, True)
    rows = N * (T0 // 8)
    flat = h.reshape(rows, CODE)
    cb = cb_ref[...]
    s = jnp.sum(flat * flat, axis=-1, keepdims=True)
    mm = jax.lax.dot_general(flat.astype(BF16), cb, (((1,), (1,)), ((), ())),
                             preferred_element_type=F32)
    d = s - 2.0 * mm + n2_ref[...]
    dmin = jnp.min(d, axis=-1, keepdims=True)
    ii = jax.lax.broadcasted_iota(jnp.int32, d.shape, 1)
    cand = jnp.where(d <= dmin, ii, NB_CODE)
    idx = jnp.min(cand, axis=-1, keepdims=True)
    oh = (ii == idx).astype(F32)
    q = _mm_hi(oh, cb)
    loss_ref[...] = jnp.mean((q - flat) ** 2).reshape(1, 1)
    qh = q.reshape(N, T0 // 8, CODE)
    qh_ref[...] = qh
    idx_ref[...] = idx
    hd_ref[...] = _relu(_csame(qh, wd0_ref, bd0_ref, 1, 0, 0, False))


def _full_spec(shape):
    return pl.BlockSpec(shape, lambda i: (0,) * len(shape))


def _batch_spec(shape, nb):
    blk = (nb,) + shape[1:]
    return pl.BlockSpec(blk, lambda i: (i,) + (0,) * (len(shape) - 1))


def _stage(body, x, weights, out_shapes, nb, multi_out=False, extra=()):
    grid = (x.shape[0] // nb,)
    in_specs = ([_batch_spec(x.shape, nb)]
                + [_batch_spec(e.shape, nb) for e in extra]
                + [_full_spec(w.shape) for w in weights])
    if multi_out:
        out_specs = [_full_spec(s.shape) for s in out_shapes]
        out_specs[0] = _batch_spec(out_shapes[0].shape, nb)
    else:
        out_specs = _batch_spec(out_shapes.shape, nb)
    return pl.pallas_call(
        body, grid=grid, in_specs=in_specs, out_specs=out_specs,
        out_shape=out_shapes)(x, *extra, *weights)


def _wt(w, dtype=F32):
    return jnp.transpose(w, (2, 1, 0)).astype(dtype)


def _pair(h):
    n, t, c = h.shape
    hp = jnp.pad(h, ((0, 0), (1, 1), (0, 0)))
    return hp.reshape(n, (t + 2) // 2, 2 * c)


def _res_weights(res):
    w3 = jnp.concatenate([_wt(c1[0]) for (c1, c2) in res], axis=0)
    b3 = jnp.stack([c1[1] for (c1, c2) in res], axis=0)
    w1 = jnp.stack([_wt(c2[0], BF16)[0] for (c1, c2) in res], axis=0)
    b1 = jnp.stack([c2[1] for (c1, c2) in res], axis=0)
    return w3, b3, w1, b1


def _xconv_d9(h_ntc, wb):
    w, b = wb
    h = h_ntc.transpose(0, 2, 1)
    r = _relu(h)
    out = jax.lax.conv_general_dilated(
        r, w, window_strides=(1,), padding=[(9, 9)], rhs_dilation=(9,),
        dimension_numbers=('NCH', 'OIH', 'NCH'))
    return (out + b[None, :, None]).transpose(0, 2, 1)


def kernel(x, enc_params, dec_params, codebook):
    x = x.astype(F32)

    w0, b0 = enc_params['conv0']
    h = jax.lax.conv_general_dilated(
        x.transpose(0, 2, 1), w0, window_strides=(1,), padding=[(1, 1)],
        dimension_numbers=('NCH', 'OIH', 'NCH'))
    h = _relu(h + b0[None, :, None]).transpose(0, 2, 1)
    t = T0
    for bi, blk in enumerate(enc_params['downs']):
        t //= 2
        wd = _wt(blk['down'][0], BF16)
        bd = blk['down'][1][None]
        w3, b3, w1, b1 = _res_weights(blk['res'])
        h2 = _stage(functools.partial(_enc_block_body, tout=t),
                    _pair(h), (wd, bd, w3, b3, w1, b1),
                    jax.ShapeDtypeStruct((N, t, W), F32), nb=max(8, 512 // t))
        r9 = _xconv_d9(h2, blk['res'][2][0])
        if bi < 2:
            h = _stage(functools.partial(_u3fin_body, c1bf=ENC_C1_BF[bi][2]),
                       h2, (w1[2:3], b1[2:3]),
                       jax.ShapeDtypeStruct((N, t, W), F32),
                       nb=max(8, 512 // t), extra=(r9,))
        else:
            w1_3, b1_3 = w1[2:3], b1[2:3]

    rows = N * t
    n2 = jnp.sum(codebook * codebook, axis=-1)[None, :]
    qh, idx, loss, h = _stage(
        _vq_body, h2,
        (w1_3, b1_3,
         _wt(enc_params['conv_out'][0]), enc_params['conv_out'][1][None],
         codebook, n2,
         _wt(dec_params['conv0'][0]), dec_params['conv0'][1][None]),
        (jax.ShapeDtypeStruct((N, t, CODE), F32),
         jax.ShapeDtypeStruct((rows, 1), jnp.int32),
         jax.ShapeDtypeStruct((1, 1), F32),
         jax.ShapeDtypeStruct((N, t, W), F32)),
        nb=N, multi_out=True, extra=(r9,))
    for blk in dec_params['ups']:
        w3, b3, w1, b1 = _res_weights(blk['res'])
        wu = _wt(blk['up_conv'][0])
        bu = blk['up_conv'][1][None]
        hp = _stage(_dec_block_body, h, (w3, b3, w1, b1, wu, bu),
                    jax.ShapeDtypeStruct((N, t, 2 * W), F32),
                    nb=max(8, 512 // t))
        t *= 2
        h = hp.reshape(N, t, W)

    out = _stage(_final_body, h,
                 (_wt(dec_params['conv1'][0]), dec_params['conv1'][1][None],
                  _wt(dec_params['conv_final'][0]),
                  dec_params['conv_final'][1][None]),
                 jax.ShapeDtypeStruct((N, T0, CIN), F32), nb=8)

    return out, idx.reshape(N, T0 // 8), loss.reshape(()), qh
